# Initial kernel scaffold; baseline (speedup 1.0000x reference)
import jax
import jax.numpy as jnp
from jax.experimental import pallas as pl


def kernel(src_embedding, tgt_embedding, src, tgt):
    raise NotImplementedError("write your pallas kernel here")


# speedup vs baseline: 4.1094x; 4.1094x over previous
import math

import jax
import jax.numpy as jnp
from jax.experimental import pallas as pl
from jax.experimental.pallas import tpu as pltpu

B, DK, N, M = 8, 64, 2048, 2048
NS = 15
K = 16
TILE_N = 256
POWER_ITERS = 320


def _pass1(se_ref, te_ref, tgt_ref, vals_ref, idx_ref, rowsum_ref, corr_ref):
    t = pl.program_id(1)
    se = se_ref[0]
    te = te_ref[0]
    logits = jax.lax.dot_general(
        se, te, (((0,), (0,)), ((), ())),
        preferred_element_type=jnp.float32,
        precision=jax.lax.Precision.HIGHEST) * (1.0 / math.sqrt(DK))
    rowmax = jnp.max(logits, axis=1, keepdims=True)
    e = jnp.exp(logits - rowmax)
    rowsum = jnp.sum(e, axis=1, keepdims=True)
    rowsum_ref[0] = rowsum

    et = jax.lax.dot_general(
        e, tgt_ref[0], (((1,), (0,)), ((), ())),
        preferred_element_type=jnp.float32,
        precision=jax.lax.Precision.HIGHEST)
    c3 = jnp.sum(et / rowsum, axis=0, keepdims=True)

    @pl.when(t == 0)
    def _():
        corr_ref[...] = jnp.zeros_like(corr_ref)
    corr_ref[0, :, 0:3] += c3

    iota_m = jax.lax.broadcasted_iota(jnp.int32, (TILE_N, M), 1)
    ew = e
    for k in range(K):
        cm = jnp.max(ew, axis=1, keepdims=True)
        ci = jnp.min(jnp.where(ew == cm, iota_m, M),
                     axis=1, keepdims=True)
        vals_ref[0, :, k:k + 1] = cm
        idx_ref[0, :, k:k + 1] = ci
        if k + 1 < K:
            ew = jnp.where(iota_m == ci, -1.0, ew)


def _pass2(vals_ref, idx_ref, rowsum_ref, src_ref, tgt_ref, corr_ref,
           stats_ref):
    vals0 = vals_ref[0] / rowsum_ref[0]
    idx = idx_ref[0]
    src2d = src_ref[0]
    tgt2d = tgt_ref[0]

    iota_n = jax.lax.broadcasted_iota(jnp.int32, (N, 1), 0)
    iota_mcol = jax.lax.broadcasted_iota(jnp.int32, (M, 1), 0)
    iota16 = jax.lax.broadcasted_iota(jnp.int32, (K, 1), 0)

    def body(i, carry):
        vals, ts, tt = carry
        bm = jnp.max(vals, axis=1, keepdims=True)
        s = jnp.max(bm)
        n = jnp.min(jnp.where(bm == s, iota_n, N))
        rowmask = iota_n == n
        vn = jnp.sum(jnp.where(rowmask, vals, 0.0), axis=0,
                     keepdims=True)
        cn = jnp.sum(jnp.where(rowmask, idx, 0), axis=0,
                     keepdims=True)
        c = jnp.min(jnp.where(vn == s, cn, M))
        srcp = jnp.sum(jnp.where(rowmask, src2d, 0.0), axis=0,
                       keepdims=True)
        tgtp = jnp.sum(jnp.where(iota_mcol == c, tgt2d, 0.0), axis=0,
                       keepdims=True)
        ts = jnp.where(iota16 == i, srcp, ts)
        tt = jnp.where(iota16 == i, tgtp, tt)
        vals = jnp.where(idx == c, -1.0, vals)
        vals = jnp.where(rowmask, -1.0, vals)
        return vals, ts, tt

    ts0 = jnp.zeros((K, 3), jnp.float32)
    _, ts, tt = jax.lax.fori_loop(0, NS, body, (vals0, ts0, ts0))

    ms = jnp.sum(ts, axis=0, keepdims=True) * (1.0 / NS)
    mt = jnp.sum(tt, axis=0, keepdims=True) * (1.0 / NS)
    valid = iota16 < NS
    tsc = jnp.where(valid, ts - ms, 0.0)
    ttc = jnp.where(valid, tt - mt, 0.0)

    sm = jnp.sum(src2d, axis=0, keepdims=True) * (1.0 / N)
    corrv = corr_ref[0, 0:1, 0:3] * (1.0 / N)

    lane = jax.lax.broadcasted_iota(jnp.int32, (1, 128), 1)
    out = jnp.zeros((1, 128), jnp.float32)
    for a in range(3):
        for b in range(3):
            sab = jnp.sum(tsc[:, a:a + 1] * ttc[:, b:b + 1], keepdims=True)
            out = jnp.where(lane == 3 * a + b, sab, out)
    for k in range(3):
        out = jnp.where(lane == 9 + k, sm[0:1, k:k + 1], out)
        out = jnp.where(lane == 12 + k, corrv[0:1, k:k + 1], out)
    stats_ref[0] = out.reshape(1, 128)


def _pass3(stats_ref, out_ref):
    st = stats_ref[...].reshape(B, 128)

    def g(j):
        return st[:, j:j + 1]

    sxx, sxy, sxz = g(0), g(1), g(2)
    syx, syy, syz = g(3), g(4), g(5)
    szx, szy, szz = g(6), g(7), g(8)

    n00 = sxx + syy + szz
    n01 = syz - szy
    n02 = szx - sxz
    n03 = sxy - syx
    n11 = sxx - syy - szz
    n12 = sxy + syx
    n13 = szx + sxz
    n22 = -sxx + syy - szz
    n23 = syz + szy
    n33 = -sxx - syy + szz

    sigma = jnp.sqrt(n00 * n00 + n11 * n11 + n22 * n22 + n33 * n33
                     + 2.0 * (n01 * n01 + n02 * n02 + n03 * n03
                              + n12 * n12 + n13 * n13 + n23 * n23))
    a00 = n00 + sigma
    a11 = n11 + sigma
    a22 = n22 + sigma
    a33 = n33 + sigma

    def piter(_, q):
        q0, q1, q2, q3 = q
        y0 = a00 * q0 + n01 * q1 + n02 * q2 + n03 * q3
        y1 = n01 * q0 + a11 * q1 + n12 * q2 + n13 * q3
        y2 = n02 * q0 + n12 * q1 + a22 * q2 + n23 * q3
        y3 = n03 * q0 + n13 * q1 + n23 * q2 + a33 * q3
        r = jax.lax.rsqrt(y0 * y0 + y1 * y1 + y2 * y2 + y3 * y3)
        return y0 * r, y1 * r, y2 * r, y3 * r

    qinit = (jnp.full((B, 1), 1.0, jnp.float32),
             jnp.full((B, 1), 0.3, jnp.float32),
             jnp.full((B, 1), 0.2, jnp.float32),
             jnp.full((B, 1), 0.1, jnp.float32))
    q0, q1, q2, q3 = jax.lax.fori_loop(0, POWER_ITERS, piter, qinit)

    r00 = q0 * q0 + q1 * q1 - q2 * q2 - q3 * q3
    r01 = 2.0 * (q1 * q2 - q0 * q3)
    r02 = 2.0 * (q1 * q3 + q0 * q2)
    r10 = 2.0 * (q2 * q1 + q0 * q3)
    r11 = q0 * q0 - q1 * q1 + q2 * q2 - q3 * q3
    r12 = 2.0 * (q2 * q3 - q0 * q1)
    r20 = 2.0 * (q3 * q1 - q0 * q2)
    r21 = 2.0 * (q3 * q2 + q0 * q1)
    r22 = q0 * q0 - q1 * q1 - q2 * q2 + q3 * q3

    sm0, sm1, sm2 = g(9), g(10), g(11)
    c0, c1, c2 = g(12), g(13), g(14)
    t0 = -(r00 * sm0 + r01 * sm1 + r02 * sm2) + c0
    t1 = -(r10 * sm0 + r11 * sm1 + r12 * sm2) + c1
    t2 = -(r20 * sm0 + r21 * sm1 + r22 * sm2) + c2

    lane = jax.lax.broadcasted_iota(jnp.int32, (B, 128), 1)
    out = jnp.zeros((B, 128), jnp.float32)
    pieces = [r00, r01, r02, r10, r11, r12, r20, r21, r22, t0, t1, t2]
    for j, v in enumerate(pieces):
        out = jnp.where(lane == j, v, out)
    out_ref[...] = out.reshape(B, 1, 128)


def kernel(src_embedding, tgt_embedding, src, tgt):
    vals, idx, rowsum, corr = pl.pallas_call(
        _pass1,
        grid=(B, N // TILE_N),
        in_specs=[
            pl.BlockSpec((1, DK, TILE_N), lambda b, t: (b, 0, t)),
            pl.BlockSpec((1, DK, M), lambda b, t: (b, 0, 0)),
            pl.BlockSpec((1, M, 3), lambda b, t: (b, 0, 0)),
        ],
        out_specs=[
            pl.BlockSpec((1, TILE_N, K), lambda b, t: (b, t, 0)),
            pl.BlockSpec((1, TILE_N, K), lambda b, t: (b, t, 0)),
            pl.BlockSpec((1, TILE_N, 1), lambda b, t: (b, t, 0)),
            pl.BlockSpec((1, 1, 8), lambda b, t: (b, 0, 0)),
        ],
        out_shape=[
            jax.ShapeDtypeStruct((B, N, K), jnp.float32),
            jax.ShapeDtypeStruct((B, N, K), jnp.int32),
            jax.ShapeDtypeStruct((B, N, 1), jnp.float32),
            jax.ShapeDtypeStruct((B, 1, 8), jnp.float32),
        ],
        compiler_params=pltpu.CompilerParams(
            dimension_semantics=("parallel", "arbitrary")),
    )(src_embedding, tgt_embedding, tgt)

    stats = pl.pallas_call(
        _pass2,
        grid=(B,),
        in_specs=[
            pl.BlockSpec((1, N, K), lambda b: (b, 0, 0)),
            pl.BlockSpec((1, N, K), lambda b: (b, 0, 0)),
            pl.BlockSpec((1, N, 1), lambda b: (b, 0, 0)),
            pl.BlockSpec((1, N, 3), lambda b: (b, 0, 0)),
            pl.BlockSpec((1, M, 3), lambda b: (b, 0, 0)),
            pl.BlockSpec((1, 1, 8), lambda b: (b, 0, 0)),
        ],
        out_specs=pl.BlockSpec((1, 1, 128), lambda b: (b, 0, 0)),
        out_shape=jax.ShapeDtypeStruct((B, 1, 128), jnp.float32),
    )(vals, idx, rowsum, src, tgt, corr)

    outb = pl.pallas_call(
        _pass3,
        grid=(1,),
        in_specs=[pl.BlockSpec((B, 1, 128), lambda i: (0, 0, 0))],
        out_specs=pl.BlockSpec((B, 1, 128), lambda i: (0, 0, 0)),
        out_shape=jax.ShapeDtypeStruct((B, 1, 128), jnp.float32),
    )(stats)

    R = outb[:, 0, 0:9].reshape(B, 3, 3)
    t = outb[:, 0, 9:12]
    return (R, t)


# speedup vs baseline: 5.5813x; 1.3582x over previous
import math

import jax
import jax.numpy as jnp
from jax.experimental import pallas as pl
from jax.experimental.pallas import tpu as pltpu

B, DK, N, M = 8, 64, 2048, 2048
NS = 15
K = 16
TILE_N = 256
POWER_ITERS = 320


def _pass1(se_ref, te_ref, tgt_ref, vals_ref, idx_ref, rowsum_ref, corr_ref):
    t = pl.program_id(1)
    se = se_ref[0]
    te = te_ref[0]
    lt = jax.lax.dot_general(
        te, se, (((0,), (0,)), ((), ())),
        preferred_element_type=jnp.float32) * (1.0 / math.sqrt(DK))
    rowmax = jnp.max(lt, axis=0, keepdims=True)
    e = jnp.exp(lt - rowmax)
    rowsum = jnp.sum(e, axis=0, keepdims=True)
    rowsum_ref[0] = rowsum

    et = jax.lax.dot_general(
        e, tgt_ref[0], (((0,), (0,)), ((), ())),
        preferred_element_type=jnp.float32,
        precision=jax.lax.Precision.HIGHEST)
    c3 = jax.lax.dot_general(
        1.0 / rowsum, et, (((1,), (0,)), ((), ())),
        preferred_element_type=jnp.float32,
        precision=jax.lax.Precision.HIGHEST)

    @pl.when(t == 0)
    def _():
        corr_ref[...] = jnp.zeros_like(corr_ref)
    corr_ref[0, :, 0:3] += c3

    iota_m = jax.lax.broadcasted_iota(jnp.int32, (M, TILE_N), 0)
    ew = e
    for k in range(NS):
        cm = jnp.max(ew, axis=0, keepdims=True)
        ci = jnp.min(jnp.where(ew == cm, iota_m, M),
                     axis=0, keepdims=True)
        vals_ref[0, k:k + 1, :] = cm
        idx_ref[0, k:k + 1, :] = ci
        if k + 1 < NS:
            ew = jnp.where(iota_m == ci, -1.0, ew)
    vals_ref[0, NS:K, :] = jnp.full((K - NS, TILE_N), -1.0, jnp.float32)
    idx_ref[0, NS:K, :] = jnp.full((K - NS, TILE_N), M, jnp.int32)


def _pass2(vals_ref, idx_ref, rowsum_ref, src_ref, tgt_ref, corr_ref,
           stats_ref):
    vals0 = vals_ref[0] / rowsum_ref[0]
    idx = idx_ref[0]
    srcT = src_ref[0]
    tgtT = tgt_ref[0]

    iota_n = jax.lax.broadcasted_iota(jnp.int32, (1, N), 1)
    iota_m = jax.lax.broadcasted_iota(jnp.int32, (1, M), 1)
    iota16 = jax.lax.broadcasted_iota(jnp.int32, (1, K), 1)

    def body(i, carry):
        vals, ts, tt = carry
        bm = jnp.max(vals, axis=0, keepdims=True)
        s = jnp.max(bm)
        n = jnp.min(jnp.where(bm == s, iota_n, N))
        colmask = iota_n == n
        vn = jnp.sum(jnp.where(colmask, vals, 0.0), axis=1,
                     keepdims=True)
        cn = jnp.sum(jnp.where(colmask, idx, 0), axis=1,
                     keepdims=True)
        c = jnp.min(jnp.where(vn == s, cn, M))
        srcp = jnp.sum(jnp.where(colmask, srcT, 0.0), axis=1,
                       keepdims=True)
        tgtp = jnp.sum(jnp.where(iota_m == c, tgtT, 0.0), axis=1,
                       keepdims=True)
        ts = jnp.where(iota16 == i, srcp, ts)
        tt = jnp.where(iota16 == i, tgtp, tt)
        vals = jnp.where(idx == c, -1.0, vals)
        vals = jnp.where(colmask, -1.0, vals)
        return vals, ts, tt

    ts0 = jnp.zeros((3, K), jnp.float32)
    _, ts, tt = jax.lax.fori_loop(0, NS, body, (vals0, ts0, ts0))

    ms = jnp.sum(ts, axis=1, keepdims=True) * (1.0 / NS)
    mt = jnp.sum(tt, axis=1, keepdims=True) * (1.0 / NS)
    valid = iota16 < NS
    tsc = jnp.where(valid, ts - ms, 0.0)
    ttc = jnp.where(valid, tt - mt, 0.0)

    sm = jnp.sum(srcT, axis=1, keepdims=True) * (1.0 / N)
    corrv = corr_ref[0, 0:1, 0:3] * (1.0 / N)

    lane = jax.lax.broadcasted_iota(jnp.int32, (1, 128), 1)
    out = jnp.zeros((1, 128), jnp.float32)
    for a in range(3):
        for b in range(3):
            sab = jnp.sum(tsc[a:a + 1, :] * ttc[b:b + 1, :], keepdims=True)
            out = jnp.where(lane == 3 * a + b, sab, out)
    for k in range(3):
        out = jnp.where(lane == 9 + k, sm[k:k + 1, 0:1], out)
        out = jnp.where(lane == 12 + k, corrv[0:1, k:k + 1], out)
    stats_ref[0] = out.reshape(1, 128)


def _pass3(stats_ref, out_ref):
    st = stats_ref[...].reshape(B, 128)

    def g(j):
        return st[:, j:j + 1]

    sxx, sxy, sxz = g(0), g(1), g(2)
    syx, syy, syz = g(3), g(4), g(5)
    szx, szy, szz = g(6), g(7), g(8)

    n00 = sxx + syy + szz
    n01 = syz - szy
    n02 = szx - sxz
    n03 = sxy - syx
    n11 = sxx - syy - szz
    n12 = sxy + syx
    n13 = szx + sxz
    n22 = -sxx + syy - szz
    n23 = syz + szy
    n33 = -sxx - syy + szz

    sigma = jnp.sqrt(n00 * n00 + n11 * n11 + n22 * n22 + n33 * n33
                     + 2.0 * (n01 * n01 + n02 * n02 + n03 * n03
                              + n12 * n12 + n13 * n13 + n23 * n23))
    a00 = n00 + sigma
    a11 = n11 + sigma
    a22 = n22 + sigma
    a33 = n33 + sigma

    def piter(_, q):
        q0, q1, q2, q3 = q
        y0 = a00 * q0 + n01 * q1 + n02 * q2 + n03 * q3
        y1 = n01 * q0 + a11 * q1 + n12 * q2 + n13 * q3
        y2 = n02 * q0 + n12 * q1 + a22 * q2 + n23 * q3
        y3 = n03 * q0 + n13 * q1 + n23 * q2 + a33 * q3
        r = jax.lax.rsqrt(y0 * y0 + y1 * y1 + y2 * y2 + y3 * y3)
        return y0 * r, y1 * r, y2 * r, y3 * r

    qinit = (jnp.full((B, 1), 1.0, jnp.float32),
             jnp.full((B, 1), 0.3, jnp.float32),
             jnp.full((B, 1), 0.2, jnp.float32),
             jnp.full((B, 1), 0.1, jnp.float32))
    q0, q1, q2, q3 = jax.lax.fori_loop(0, POWER_ITERS, piter, qinit)

    r00 = q0 * q0 + q1 * q1 - q2 * q2 - q3 * q3
    r01 = 2.0 * (q1 * q2 - q0 * q3)
    r02 = 2.0 * (q1 * q3 + q0 * q2)
    r10 = 2.0 * (q2 * q1 + q0 * q3)
    r11 = q0 * q0 - q1 * q1 + q2 * q2 - q3 * q3
    r12 = 2.0 * (q2 * q3 - q0 * q1)
    r20 = 2.0 * (q3 * q1 - q0 * q2)
    r21 = 2.0 * (q3 * q2 + q0 * q1)
    r22 = q0 * q0 - q1 * q1 - q2 * q2 + q3 * q3

    sm0, sm1, sm2 = g(9), g(10), g(11)
    c0, c1, c2 = g(12), g(13), g(14)
    t0 = -(r00 * sm0 + r01 * sm1 + r02 * sm2) + c0
    t1 = -(r10 * sm0 + r11 * sm1 + r12 * sm2) + c1
    t2 = -(r20 * sm0 + r21 * sm1 + r22 * sm2) + c2

    lane = jax.lax.broadcasted_iota(jnp.int32, (B, 128), 1)
    out = jnp.zeros((B, 128), jnp.float32)
    pieces = [r00, r01, r02, r10, r11, r12, r20, r21, r22, t0, t1, t2]
    for j, v in enumerate(pieces):
        out = jnp.where(lane == j, v, out)
    out_ref[...] = out.reshape(B, 1, 128)


def kernel(src_embedding, tgt_embedding, src, tgt):
    src_t = jnp.transpose(src, (0, 2, 1))
    tgt_t = jnp.transpose(tgt, (0, 2, 1))

    vals, idx, rowsum, corr = pl.pallas_call(
        _pass1,
        grid=(B, N // TILE_N),
        in_specs=[
            pl.BlockSpec((1, DK, TILE_N), lambda b, t: (b, 0, t)),
            pl.BlockSpec((1, DK, M), lambda b, t: (b, 0, 0)),
            pl.BlockSpec((1, M, 3), lambda b, t: (b, 0, 0)),
        ],
        out_specs=[
            pl.BlockSpec((1, K, TILE_N), lambda b, t: (b, 0, t)),
            pl.BlockSpec((1, K, TILE_N), lambda b, t: (b, 0, t)),
            pl.BlockSpec((1, 1, TILE_N), lambda b, t: (b, 0, t)),
            pl.BlockSpec((1, 1, 8), lambda b, t: (b, 0, 0)),
        ],
        out_shape=[
            jax.ShapeDtypeStruct((B, K, N), jnp.float32),
            jax.ShapeDtypeStruct((B, K, N), jnp.int32),
            jax.ShapeDtypeStruct((B, 1, N), jnp.float32),
            jax.ShapeDtypeStruct((B, 1, 8), jnp.float32),
        ],
        compiler_params=pltpu.CompilerParams(
            dimension_semantics=("parallel", "arbitrary")),
    )(src_embedding, tgt_embedding, tgt)

    stats = pl.pallas_call(
        _pass2,
        grid=(B,),
        in_specs=[
            pl.BlockSpec((1, K, N), lambda b: (b, 0, 0)),
            pl.BlockSpec((1, K, N), lambda b: (b, 0, 0)),
            pl.BlockSpec((1, 1, N), lambda b: (b, 0, 0)),
            pl.BlockSpec((1, 3, N), lambda b: (b, 0, 0)),
            pl.BlockSpec((1, 3, M), lambda b: (b, 0, 0)),
            pl.BlockSpec((1, 1, 8), lambda b: (b, 0, 0)),
        ],
        out_specs=pl.BlockSpec((1, 1, 128), lambda b: (b, 0, 0)),
        out_shape=jax.ShapeDtypeStruct((B, 1, 128), jnp.float32),
    )(vals, idx, rowsum, src_t, tgt_t, corr)

    outb = pl.pallas_call(
        _pass3,
        grid=(1,),
        in_specs=[pl.BlockSpec((B, 1, 128), lambda i: (0, 0, 0))],
        out_specs=pl.BlockSpec((B, 1, 128), lambda i: (0, 0, 0)),
        out_shape=jax.ShapeDtypeStruct((B, 1, 128), jnp.float32),
    )(stats)

    R = outb[:, 0, 0:9].reshape(B, 3, 3)
    t = outb[:, 0, 9:12]
    return (R, t)
